# baseline (device time: 39460 ns/iter reference)
import functools

import jax
import jax.numpy as jnp
from jax import lax
from jax.experimental import pallas as pl
from jax.experimental.pallas import tpu as pltpu

N_DEV = 8
SEQ = 1024
HQ = 8
DH = 128
WIN = 128
D_MODEL = HQ * DH
SCALE = 0.08838834764831843


def kernel(x, Wq, K_ext, V_ext, Wo):
    xb = x.reshape(SEQ, D_MODEL).astype(jnp.bfloat16)
    wq = Wq.astype(jnp.bfloat16)
    wo = Wo.astype(jnp.bfloat16)
    kt = K_ext.reshape(SEQ, HQ, DH).transpose(1, 0, 2).astype(jnp.bfloat16)
    vt = V_ext.reshape(SEQ, HQ, DH).transpose(1, 0, 2).astype(jnp.bfloat16)

    def body(x_ref, wq_ref, k_ref, v_ref, wo_ref, out_ref,
             sendl, sendr, halol, halor, q_scr, ctx_scr,
             send_sems, recv_sems):
        my = lax.axis_index("i")
        left = lax.rem(my + N_DEV - 1, N_DEV)
        right = lax.rem(my + 1, N_DEV)

        barrier_sem = pltpu.get_barrier_semaphore()
        for nbr in (left, right):
            pl.semaphore_signal(
                barrier_sem, inc=1,
                device_id=(nbr,), device_id_type=pl.DeviceIdType.MESH,
            )
        pl.semaphore_wait(barrier_sem, 2)

        sendr[0] = k_ref[:, SEQ - WIN:, :]
        sendr[1] = v_ref[:, SEQ - WIN:, :]
        sendl[0] = k_ref[:, :WIN, :]
        sendl[1] = v_ref[:, :WIN, :]

        rdma_r = pltpu.make_async_remote_copy(
            src_ref=sendr, dst_ref=halol,
            send_sem=send_sems.at[1], recv_sem=recv_sems.at[0],
            device_id=(right,), device_id_type=pl.DeviceIdType.MESH,
        )
        rdma_l = pltpu.make_async_remote_copy(
            src_ref=sendl, dst_ref=halor,
            send_sem=send_sems.at[0], recv_sem=recv_sems.at[1],
            device_id=(left,), device_id_type=pl.DeviceIdType.MESH,
        )
        rdma_r.start()
        rdma_l.start()

        q = jnp.dot(x_ref[:], wq_ref[:], preferred_element_type=jnp.float32)
        q_scr[:] = (q * SCALE).astype(jnp.bfloat16)

        rdma_r.wait_recv()
        rdma_l.wait_recv()
        rdma_r.wait_send()
        rdma_l.wait_send()

        kv = SEQ + 2 * WIN
        qi = lax.broadcasted_iota(jnp.int32, (SEQ, kv), 0)
        ki = lax.broadcasted_iota(jnp.int32, (SEQ, kv), 1)
        delta = qi - ki + WIN
        ki_glob = my * SEQ - WIN + ki
        mask = ((delta >= -WIN) & (delta <= WIN)
                & (ki_glob >= 0) & (ki_glob < N_DEV * SEQ))
        neg = jnp.float32(-1e9)

        for h in range(HQ):
            kh = jnp.concatenate(
                [halol[0, h], k_ref[h], halor[0, h]], axis=0)
            vh = jnp.concatenate(
                [halol[1, h], v_ref[h], halor[1, h]], axis=0)
            qh = q_scr[:, h * DH:(h + 1) * DH]
            scores = lax.dot_general(
                qh, kh, (((1,), (1,)), ((), ())),
                preferred_element_type=jnp.float32,
            )
            scores = jnp.where(mask, scores, neg)
            m = jnp.max(scores, axis=1, keepdims=True)
            p = jnp.exp(scores - m)
            s = jnp.sum(p, axis=1, keepdims=True)
            w = (p / s).astype(jnp.bfloat16)
            ctx = jnp.dot(w, vh, preferred_element_type=jnp.float32)
            ctx_scr[:, h * DH:(h + 1) * DH] = ctx.astype(jnp.bfloat16)

        out_ref[:] = jnp.dot(ctx_scr[:], wo_ref[:],
                             preferred_element_type=jnp.float32)

    out = pl.pallas_call(
        body,
        out_shape=jax.ShapeDtypeStruct((SEQ, D_MODEL), jnp.float32),
        in_specs=[pl.BlockSpec(memory_space=pltpu.VMEM)] * 5,
        out_specs=pl.BlockSpec(memory_space=pltpu.VMEM),
        scratch_shapes=[
            pltpu.VMEM((2, HQ, WIN, DH), jnp.bfloat16),
            pltpu.VMEM((2, HQ, WIN, DH), jnp.bfloat16),
            pltpu.VMEM((2, HQ, WIN, DH), jnp.bfloat16),
            pltpu.VMEM((2, HQ, WIN, DH), jnp.bfloat16),
            pltpu.VMEM((SEQ, D_MODEL), jnp.bfloat16),
            pltpu.VMEM((SEQ, D_MODEL), jnp.bfloat16),
            pltpu.SemaphoreType.DMA((2,)),
            pltpu.SemaphoreType.DMA((2,)),
        ],
        compiler_params=pltpu.CompilerParams(collective_id=0),
    )(xb, wq, kt, vt, wo)
    return out.reshape(1, SEQ, D_MODEL)


# device time: 32363 ns/iter; 1.2193x vs baseline; 1.2193x over previous
import jax
import jax.numpy as jnp
from jax import lax
from jax.experimental import pallas as pl
from jax.experimental.pallas import tpu as pltpu

N_DEV = 8
SEQ = 1024
HQ = 8
DH = 128
WIN = 128
D_MODEL = HQ * DH
QBLK = 256
KBLK = QBLK + 2 * WIN
SCALE = 0.08838834764831843


def kernel(x, Wq, K_ext, V_ext, Wo):
    xb = x.reshape(SEQ, D_MODEL)
    kb = K_ext.reshape(SEQ, D_MODEL)
    vb = V_ext.reshape(SEQ, D_MODEL)

    def body(x_ref, wq_ref, k_ref, v_ref, wo_ref, out_ref,
             sendl, sendr, halol, halor, k_full, v_full, ctx_scr,
             send_sems, recv_sems):
        my = lax.axis_index("i")
        left = lax.rem(my + N_DEV - 1, N_DEV)
        right = lax.rem(my + 1, N_DEV)

        barrier_sem = pltpu.get_barrier_semaphore()
        for nbr in (left, right):
            pl.semaphore_signal(
                barrier_sem, inc=1,
                device_id=(nbr,), device_id_type=pl.DeviceIdType.MESH,
            )
        pl.semaphore_wait(barrier_sem, 2)

        sendr[0] = k_ref[SEQ - WIN:, :].astype(jnp.bfloat16)
        sendr[1] = v_ref[SEQ - WIN:, :].astype(jnp.bfloat16)
        sendl[0] = k_ref[:WIN, :].astype(jnp.bfloat16)
        sendl[1] = v_ref[:WIN, :].astype(jnp.bfloat16)

        rdma_r = pltpu.make_async_remote_copy(
            src_ref=sendr, dst_ref=halol,
            send_sem=send_sems.at[1], recv_sem=recv_sems.at[0],
            device_id=(right,), device_id_type=pl.DeviceIdType.MESH,
        )
        rdma_l = pltpu.make_async_remote_copy(
            src_ref=sendl, dst_ref=halor,
            send_sem=send_sems.at[0], recv_sem=recv_sems.at[1],
            device_id=(left,), device_id_type=pl.DeviceIdType.MESH,
        )
        rdma_r.start()
        rdma_l.start()

        q_bf = (jnp.dot(x_ref[:].astype(jnp.bfloat16),
                        wq_ref[:].astype(jnp.bfloat16),
                        preferred_element_type=jnp.float32)
                * SCALE).astype(jnp.bfloat16)
        k_full[WIN:WIN + SEQ, :] = k_ref[:].astype(jnp.bfloat16)
        v_full[WIN:WIN + SEQ, :] = v_ref[:].astype(jnp.bfloat16)

        rdma_r.wait_recv()
        rdma_l.wait_recv()
        k_full[:WIN, :] = halol[0]
        v_full[:WIN, :] = halol[1]
        k_full[WIN + SEQ:, :] = halor[0]
        v_full[WIN + SEQ:, :] = halor[1]
        rdma_r.wait_send()
        rdma_l.wait_send()

        qi = lax.broadcasted_iota(jnp.int32, (QBLK, KBLK), 0)
        ki = lax.broadcasted_iota(jnp.int32, (QBLK, KBLK), 1)
        delta = qi - ki + WIN
        window = (delta >= -WIN) & (delta <= WIN)
        neg = jnp.float32(-1e9)

        for b in range(SEQ // QBLK):
            ki_glob = my * SEQ + b * QBLK - WIN + ki
            maskb = window & (ki_glob >= 0) & (ki_glob < N_DEV * SEQ)
            for h in range(HQ):
                qh = q_bf[b * QBLK:(b + 1) * QBLK, h * DH:(h + 1) * DH]
                kh = k_full[b * QBLK:b * QBLK + KBLK, h * DH:(h + 1) * DH]
                scores = lax.dot_general(
                    qh, kh, (((1,), (1,)), ((), ())),
                    preferred_element_type=jnp.float32,
                )
                p = jnp.exp(jnp.where(maskb, scores, neg))
                s = jnp.sum(p, axis=1, keepdims=True)
                ctx = jnp.dot(
                    p.astype(jnp.bfloat16),
                    v_full[b * QBLK:b * QBLK + KBLK, h * DH:(h + 1) * DH],
                    preferred_element_type=jnp.float32,
                )
                ctx_scr[b * QBLK:(b + 1) * QBLK, h * DH:(h + 1) * DH] = (
                    (ctx / s).astype(jnp.bfloat16))

        out_ref[:] = jnp.dot(ctx_scr[:], wo_ref[:].astype(jnp.bfloat16),
                             preferred_element_type=jnp.float32)

    out = pl.pallas_call(
        body,
        out_shape=jax.ShapeDtypeStruct((SEQ, D_MODEL), jnp.float32),
        in_specs=[pl.BlockSpec(memory_space=pltpu.VMEM)] * 5,
        out_specs=pl.BlockSpec(memory_space=pltpu.VMEM),
        scratch_shapes=[
            pltpu.VMEM((2, WIN, D_MODEL), jnp.bfloat16),
            pltpu.VMEM((2, WIN, D_MODEL), jnp.bfloat16),
            pltpu.VMEM((2, WIN, D_MODEL), jnp.bfloat16),
            pltpu.VMEM((2, WIN, D_MODEL), jnp.bfloat16),
            pltpu.VMEM((SEQ + 2 * WIN, D_MODEL), jnp.bfloat16),
            pltpu.VMEM((SEQ + 2 * WIN, D_MODEL), jnp.bfloat16),
            pltpu.VMEM((SEQ, D_MODEL), jnp.bfloat16),
            pltpu.SemaphoreType.DMA((2,)),
            pltpu.SemaphoreType.DMA((2,)),
        ],
        compiler_params=pltpu.CompilerParams(collective_id=0),
    )(xb, Wq, kb, vb, Wo)
    return out.reshape(1, SEQ, D_MODEL)
